# baseline (speedup 1.0000x reference)
import jax
import jax.numpy as jnp
from jax import lax
from jax.experimental import pallas as pl
from jax.experimental.pallas import tpu as pltpu
from jax.experimental.pallas import tpu_sc as plsc

_NC = 2
_NS = 16
_NW = _NC * _NS
_CH = 128


def kernel(x, W):
    batch, fields = x.shape
    n = batch * fields
    d = W.shape[1]
    idx = x.reshape(n // _CH, _CH // 2, 2).transpose(0, 2, 1).reshape(n)
    Wp = jnp.pad(W, ((0, 0), (0, 128 - d)))
    b_per_w = n // _NW
    ng = b_per_w // _CH
    mesh = plsc.VectorSubcoreMesh(core_axis_name="c", subcore_axis_name="s")

    @pl.kernel(
        out_type=jax.ShapeDtypeStruct((n // 2, 128), W.dtype),
        mesh=mesh,
        scratch_types=[
            pltpu.VMEM((b_per_w,), jnp.int32),
            pltpu.VMEM((_CH, 128), W.dtype),
            pltpu.VMEM((_CH, 128), W.dtype),
            pltpu.SemaphoreType.DMA,
            pltpu.SemaphoreType.DMA,
        ],
    )
    def gather_kernel(w_hbm, i_hbm, o_hbm, idx_v, buf0, buf1, sem0, sem1):
        wid = lax.axis_index("s") * _NC + lax.axis_index("c")
        base = wid * b_per_w
        pltpu.sync_copy(i_hbm.at[pl.ds(base, b_per_w)], idx_v)

        def start(g, buf, sem):
            pltpu.async_copy(
                w_hbm.at[idx_v.at[pl.ds(g * _CH, _CH)]], buf, sem
            )

        def wait(buf, sem):
            pltpu.make_async_copy(
                w_hbm.at[pl.ds(0, _CH)], buf, sem
            ).wait()

        def pack(buf):
            @pl.loop(0, _CH // 2)
            def _(j):
                for k in range(4):
                    v = buf[64 + j, pl.ds(k * 16, 16)]
                    buf[j, pl.ds(64 + k * 16, 16)] = v

        def writeout(g, buf):
            off = pl.multiple_of((base + g * _CH) // 2, 64)
            pltpu.sync_copy(
                buf.at[pl.ds(0, _CH // 2)],
                o_hbm.at[pl.ds(off, _CH // 2)],
            )

        start(0, buf0, sem0)
        start(1, buf1, sem1)

        @pl.loop(0, ng, step=2)
        def _(g):
            wait(buf0, sem0)
            pack(buf0)
            writeout(g, buf0)

            @pl.when(g + 2 < ng)
            def _():
                start(g + 2, buf0, sem0)

            wait(buf1, sem1)
            pack(buf1)
            writeout(g + 1, buf1)

            @pl.when(g + 3 < ng)
            def _():
                start(g + 3, buf1, sem1)

    out = gather_kernel(Wp, idx)
    return out.reshape(batch, fields, d)


# speedup vs baseline: 1.0751x; 1.0751x over previous
import jax
import jax.numpy as jnp
from jax import lax
from jax.experimental import pallas as pl
from jax.experimental.pallas import tpu as pltpu
from jax.experimental.pallas import tpu_sc as plsc

_NC = 2
_NS = 16
_NW = _NC * _NS
_CH = 128
_NB = 4


def kernel(x, W):
    batch, fields = x.shape
    n = batch * fields
    d = W.shape[1]
    idx = x.reshape(n)
    Wp = jnp.pad(W, ((0, 0), (0, 128 - d)))
    b_per_w = n // _NW
    ng = b_per_w // _CH
    mesh = plsc.VectorSubcoreMesh(core_axis_name="c", subcore_axis_name="s")

    @pl.kernel(
        out_type=jax.ShapeDtypeStruct((n, 128), W.dtype),
        mesh=mesh,
        scratch_types=[
            pltpu.VMEM((b_per_w,), jnp.int32),
            pltpu.VMEM((_NB * _CH, 128), W.dtype),
            pltpu.SemaphoreType.DMA((_NB,)),
            pltpu.SemaphoreType.DMA((_NB,)),
        ],
    )
    def gather_kernel(w_hbm, i_hbm, o_hbm, idx_v, bufs, gsem, wsem):
        wid = lax.axis_index("s") * _NC + lax.axis_index("c")
        base = wid * b_per_w
        pltpu.sync_copy(i_hbm.at[pl.ds(base, b_per_w)], idx_v)

        def start(g, s):
            pltpu.async_copy(
                w_hbm.at[idx_v.at[pl.ds(g * _CH, _CH)]],
                bufs.at[pl.ds(s * _CH, _CH)],
                gsem.at[s],
            )

        def wait_gather(s):
            pltpu.make_async_copy(
                w_hbm.at[pl.ds(0, _CH)],
                bufs.at[pl.ds(s * _CH, _CH)],
                gsem.at[s],
            ).wait()

        def start_write(g, s):
            pltpu.async_copy(
                bufs.at[pl.ds(s * _CH, _CH)],
                o_hbm.at[pl.ds(base + g * _CH, _CH)],
                wsem.at[s],
            )

        def wait_write(s):
            pltpu.make_async_copy(
                bufs.at[pl.ds(s * _CH, _CH)],
                o_hbm.at[pl.ds(0, _CH)],
                wsem.at[s],
            ).wait()

        for s in range(_NB):
            start(s, s)

        @pl.loop(0, ng, step=_NB)
        def _(g):
            for s in range(_NB):
                wait_gather(s)
                start_write(g + s, s)

                @pl.when(g + s + _NB < ng)
                def _():
                    wait_write(s)
                    start(g + s + _NB, s)

        for s in range(_NB):
            wait_write(s)

    out = gather_kernel(Wp, idx)
    return out[:, :d].reshape(batch, fields, d)
